# baseline (speedup 1.0000x reference)
import jax
import jax.numpy as jnp
from jax import lax
from jax.experimental import pallas as pl
from jax.experimental.pallas import tpu as pltpu

_N = 100
_NUM_IDX = 25
_R = 32


def _rotl(x, r):
    return (x << jnp.uint32(r)) | (x >> jnp.uint32(32 - r))


def _threefry2x32(k0, k1, x0, x1):
    ks = [k0, k1, k0 ^ k1 ^ jnp.uint32(0x1BD11BDA)]
    x0 = x0 + ks[0]
    x1 = x1 + ks[1]
    rotations = ((13, 15, 26, 6), (17, 29, 16, 24))
    for i in range(5):
        for r in rotations[i % 2]:
            x0 = x0 + x1
            x1 = _rotl(x1, r)
            x1 = x0 ^ x1
        x0 = x0 + ks[(i + 1) % 3]
        x1 = x1 + ks[(i + 2) % 3] + jnp.uint32(i + 1)
    return x0, x1


def _mask_body(out_ref):
    row = lax.broadcasted_iota(jnp.int32, (_R, 128), 0)
    zero_u = jnp.zeros((_R, 128), jnp.uint32)

    k0, k1 = _threefry2x32(zero_u, zero_u, zero_u, zero_u + jnp.uint32(1))

    b0, b1 = _threefry2x32(k0, k1, zero_u, row.astype(jnp.uint32))
    bits = b0 ^ b1

    f = lax.bitcast_convert_type(
        (bits >> jnp.uint32(9)) | jnp.uint32(0x3F800000), jnp.float32)
    idx = ((f - jnp.float32(1.0)) * jnp.float32(_N)).astype(jnp.int32)

    col = lax.broadcasted_iota(jnp.int32, (_R, 128), 1)
    hit = jnp.logical_and(idx == col, row < _NUM_IDX)
    mask = jnp.any(hit, axis=0)
    out_ref[...] = mask[:_N]


def kernel(x):
    del x
    return pl.pallas_call(
        _mask_body,
        out_shape=jax.ShapeDtypeStruct((_N,), jnp.bool_),
        compiler_params=pltpu.CompilerParams(skip_device_barrier=True),
    )()
